# Initial kernel scaffold; baseline (speedup 1.0000x reference)
import jax
import jax.numpy as jnp
from jax.experimental import pallas as pl


def kernel(pred_deltas, pred_labels, prior_boxes):
    raise NotImplementedError("write your pallas kernel here")


# speedup vs baseline: 7.7770x; 7.7770x over previous
import functools

import jax
import jax.numpy as jnp
from jax.experimental import pallas as pl
from jax.experimental.pallas import tpu as pltpu

_VAR0, _VAR1, _VAR2, _VAR3 = 0.1, 0.1, 0.2, 0.2
_K = 200
_SCORE_T = 0.5
_IOU_T = 0.5
_EPS = 1e-9


def _nms_body(labels_ref, deltas_ref, priors_ref,
              sc_ref, vd_ref, b1_ref, b2_ref, b3_ref, b4_ref,
              scores_ref, y1_ref, x1_ref, y2_ref, x2_ref, area_ref):
    B, C, N = labels_ref.shape
    labels = labels_ref[...]
    mx_all = jnp.max(labels, axis=1)
    keep = mx_all > labels[:, 0, :]
    scores_ref[...] = jnp.where(
        keep[:, None, :] & (labels > _SCORE_T), labels, -1.0)

    p = priors_ref[...]
    anc_h = p[2:3, :] - p[0:1, :]
    anc_w = p[3:4, :] - p[1:2, :]
    anc_cy = p[0:1, :] + 0.5 * anc_h
    anc_cx = p[1:2, :] + 0.5 * anc_w
    d = deltas_ref[...]
    d0 = d[:, 0, :] * _VAR0
    d1 = d[:, 1, :] * _VAR1
    d2 = d[:, 2, :] * _VAR2
    d3 = d[:, 3, :] * _VAR3
    bh = jnp.exp(d2) * anc_h
    bw = jnp.exp(d3) * anc_w
    cy = d0 * anc_h + anc_cy
    cx = d1 * anc_w + anc_cx
    y1 = cy - 0.5 * bh
    x1 = cx - 0.5 * bw
    y2 = y1 + bh
    x2 = x1 + bw
    y1_ref[...] = y1
    x1_ref[...] = x1
    y2_ref[...] = y2
    x2_ref[...] = x2
    area_ref[...] = (jnp.maximum(y2 - y1, 0.0) * jnp.maximum(x2 - x1, 0.0))

    iota = jax.lax.broadcasted_iota(jnp.int32, (B, C, N), 2)
    big = jnp.int32(N)

    def step(t, carry):
        s = scores_ref[...]
        m = jnp.max(s, axis=2)
        cand = jnp.where(s == m[:, :, None], iota, big)
        idx = jnp.min(cand, axis=2)
        valid = m > _SCORE_T
        oh = (iota == idx[:, :, None]).astype(jnp.float32)
        by1v = y1_ref[...][:, None, :]
        bx1v = x1_ref[...][:, None, :]
        by2v = y2_ref[...][:, None, :]
        bx2v = x2_ref[...][:, None, :]
        wy1 = jnp.sum(oh * by1v, axis=2)
        wx1 = jnp.sum(oh * bx1v, axis=2)
        wy2 = jnp.sum(oh * by2v, axis=2)
        wx2 = jnp.sum(oh * bx2v, axis=2)
        a1 = (jnp.maximum(wy2 - wy1, 0.0) * jnp.maximum(wx2 - wx1, 0.0))
        yy1 = jnp.maximum(wy1[:, :, None], by1v)
        xx1 = jnp.maximum(wx1[:, :, None], bx1v)
        yy2 = jnp.minimum(wy2[:, :, None], by2v)
        xx2 = jnp.minimum(wx2[:, :, None], bx2v)
        inter = jnp.maximum(yy2 - yy1, 0.0) * jnp.maximum(xx2 - xx1, 0.0)
        iou = inter / (a1[:, :, None] + area_ref[...][:, None, :] - inter + _EPS)
        scores_ref[...] = jnp.where(iou >= _IOU_T, -1.0, s)
        sc_ref[pl.ds(t, 1)] = jnp.where(valid, m, 0.0)[None]
        vd_ref[pl.ds(t, 1)] = valid.astype(jnp.float32)[None]
        b1_ref[pl.ds(t, 1)] = wy1[None]
        b2_ref[pl.ds(t, 1)] = wx1[None]
        b3_ref[pl.ds(t, 1)] = wy2[None]
        b4_ref[pl.ds(t, 1)] = wx2[None]
        return carry

    jax.lax.fori_loop(0, _K, step, 0)


def _merge_body(sc_ref, vd_ref, b1_ref, b2_ref, b3_ref, b4_ref,
                so_ref, co_ref, o1_ref, o2_ref, o3_ref, o4_ref, cnt_ref,
                s_scr):
    B, M = sc_ref.shape
    s_scr[...] = sc_ref[...]
    cnt_ref[...] = jnp.zeros_like(cnt_ref)
    zero_bk = jnp.zeros((B, _K), jnp.float32)
    so_ref[...] = zero_bk
    co_ref[...] = zero_bk
    o1_ref[...] = zero_bk
    o2_ref[...] = zero_bk
    o3_ref[...] = zero_bk
    o4_ref[...] = zero_bk
    iota = jax.lax.broadcasted_iota(jnp.int32, (B, M), 1)
    kiota = jax.lax.broadcasted_iota(jnp.int32, (B, _K), 1)
    big = jnp.int32(M)

    def step(k, carry):
        s = s_scr[...]
        m = jnp.max(s, axis=1, keepdims=True)
        cand = jnp.where(s == m, iota, big)
        j = jnp.min(cand, axis=1, keepdims=True)
        ohb = iota == j
        oh = ohb.astype(jnp.float32)
        vd = jnp.sum(oh * vd_ref[...], axis=1, keepdims=True)
        cls = (j // _K).astype(jnp.float32) * vd
        w1 = jnp.sum(oh * b1_ref[...], axis=1, keepdims=True) * vd
        w2 = jnp.sum(oh * b2_ref[...], axis=1, keepdims=True) * vd
        w3 = jnp.sum(oh * b3_ref[...], axis=1, keepdims=True) * vd
        w4 = jnp.sum(oh * b4_ref[...], axis=1, keepdims=True) * vd
        kmask = kiota == k
        so_ref[...] += jnp.where(kmask, m * vd, 0.0)
        co_ref[...] += jnp.where(kmask, cls, 0.0)
        o1_ref[...] += jnp.where(kmask, w1, 0.0)
        o2_ref[...] += jnp.where(kmask, w2, 0.0)
        o3_ref[...] += jnp.where(kmask, w3, 0.0)
        o4_ref[...] += jnp.where(kmask, w4, 0.0)
        cnt_ref[...] = cnt_ref[...] + vd
        s_scr[...] = jnp.where(ohb, -2.0, s)
        return carry

    jax.lax.fori_loop(0, _K, step, 0)


@jax.jit
def kernel(pred_deltas, pred_labels, prior_boxes):
    B, N, C = pred_labels.shape
    f32 = jnp.float32
    labels_t = pred_labels.transpose(0, 2, 1)
    deltas_t = pred_deltas.transpose(0, 2, 1)
    priors_t = prior_boxes.T

    out_kc = jax.ShapeDtypeStruct((_K, B, C), f32)
    sc, vd, b1, b2, b3, b4 = pl.pallas_call(
        _nms_body,
        out_shape=(out_kc,) * 6,
        scratch_shapes=[
            pltpu.VMEM((B, C, N), f32),
            pltpu.VMEM((B, N), f32),
            pltpu.VMEM((B, N), f32),
            pltpu.VMEM((B, N), f32),
            pltpu.VMEM((B, N), f32),
            pltpu.VMEM((B, N), f32),
        ],
    )(labels_t, deltas_t, priors_t)

    def _flat(x):
        return x.transpose(1, 2, 0).reshape(B, C * _K)

    out_bk = jax.ShapeDtypeStruct((B, _K), f32)
    so, co, o1, o2, o3, o4, cnt = pl.pallas_call(
        _merge_body,
        out_shape=(out_bk,) * 6 + (jax.ShapeDtypeStruct((B, 1), f32),),
        scratch_shapes=[pltpu.VMEM((B, C * _K), f32)],
    )(_flat(sc), _flat(vd), _flat(b1), _flat(b2), _flat(b3), _flat(b4))

    nmsed_boxes = jnp.stack([o1, o2, o3, o4], axis=-1)
    nmsed_scores = so
    nmsed_classes = co
    valid_detections = cnt.reshape(B).astype(jnp.int32)
    return nmsed_boxes, nmsed_scores, nmsed_classes, valid_detections
